# Initial kernel scaffold; baseline (speedup 1.0000x reference)
import jax
import jax.numpy as jnp
from jax.experimental import pallas as pl


def kernel(sub_embed, obj_embed, rela, diag):
    raise NotImplementedError("write your pallas kernel here")


# speedup vs baseline: 1.1226x; 1.1226x over previous
import functools

import jax
import jax.numpy as jnp
from jax import lax
from jax.experimental import pallas as pl
from jax.experimental.pallas import tpu as pltpu
from jax.experimental.pallas import tpu_sc as plsc

B = 16384
D = 64
NC, NS, L = 2, 16, 16
NW = NC * NS
BPW = B // NW
IDX_CH = 128
NCH = BPW // IDX_CH
ROW_UNROLL = 4

_mesh = plsc.VectorSubcoreMesh(core_axis_name="c", subcore_axis_name="s")


@functools.partial(
    pl.kernel,
    mesh=_mesh,
    out_type=jax.ShapeDtypeStruct((B,), jnp.float32),
    scratch_types=[
        pltpu.VMEM((NCH, IDX_CH), jnp.int32),
        pltpu.VMEM((BPW, D), jnp.float32),
        pltpu.VMEM((BPW, D), jnp.float32),
        pltpu.VMEM((BPW, D), jnp.float32),
        pltpu.VMEM((BPW,), jnp.float32),
        pltpu.SemaphoreType.DMA,
        pltpu.SemaphoreType.DMA,
    ],
    compiler_params=pltpu.CompilerParams(use_tc_tiling_on_sc=False),
)
def _distmult_sc(sub_hbm, obj_hbm, rela_hbm, diag_hbm, out_hbm,
                 idx_v, sub_v, obj_v, rel_v, out_v, gsem, dsem):
    wid = lax.axis_index("s") * NC + lax.axis_index("c")
    base = wid * BPW

    pltpu.sync_copy(rela_hbm.at[pl.ds(wid * NCH, NCH), :], idx_v)

    gathers = [
        pltpu.async_copy(
            diag_hbm.at[idx_v.at[j]],
            rel_v.at[pl.ds(j * IDX_CH, IDX_CH), :],
            gsem,
        )
        for j in range(NCH)
    ]
    sub_cp = pltpu.async_copy(sub_hbm.at[pl.ds(base, BPW), :], sub_v, dsem)
    obj_cp = pltpu.async_copy(obj_hbm.at[pl.ds(base, BPW), :], obj_v, dsem)
    for g in gathers:
        g.wait()
    sub_cp.wait()
    obj_cp.wait()

    lanes = lax.iota(jnp.int32, L)
    perms = [jnp.bitwise_xor(lanes, k) for k in (8, 4, 2, 1)]

    def lanesum(v):
        for p in perms:
            v = v + v.at[p].get(mode="promise_in_bounds", unique_indices=True)
        return v

    def row(i):
        acc = None
        for c in range(D // L):
            s = sub_v[i, pl.ds(c * L, L)]
            o = obj_v[i, pl.ds(c * L, L)]
            r = rel_v[i, pl.ds(c * L, L)]
            p = (s * o) * r
            acc = p if acc is None else acc + p
        return lanesum(acc)

    def body(k, carry):
        i0 = k * L
        outvec = row(i0)
        for u in range(1, L):
            outvec = jnp.where(lanes == u, row(i0 + u), outvec)
        out_v[pl.ds(i0, L)] = outvec
        return carry

    lax.fori_loop(0, BPW // L, body, 0)

    pltpu.sync_copy(out_v, out_hbm.at[pl.ds(base, BPW)])


def kernel(sub_embed, obj_embed, rela, diag):
    rela2d = rela.astype(jnp.int32).reshape(B // IDX_CH, IDX_CH)
    return _distmult_sc(sub_embed, obj_embed, rela2d, diag)
